# baseline (speedup 1.0000x reference)
import jax
import jax.numpy as jnp
from jax.experimental import pallas as pl
from jax.experimental.pallas import tpu as pltpu

_BM = 512
_STREAMS = 2


def _router_block(x0_ref, x1_ref, w_ref, o_ref):
    for s, x_ref in enumerate((x0_ref, x1_ref)):
        o_ref[s * _BM:(s + 1) * _BM, :] = jax.lax.dot_general(
            x_ref[...], w_ref[...],
            dimension_numbers=(((1,), (1,)), ((), ())),
            preferred_element_type=jnp.float32,
            precision=jax.lax.Precision.DEFAULT,
        )


def kernel(hidden_states, W):
    hs = hidden_states.reshape(-1, hidden_states.shape[-1])
    m, k = hs.shape
    n = W.shape[0]
    bo = _BM * _STREAMS

    def stripe(s):
        return pl.BlockSpec((_BM, k), lambda i, s=s: (i * _STREAMS + s, 0))

    return pl.pallas_call(
        _router_block,
        grid=(m // bo,),
        in_specs=[stripe(0), stripe(1),
                  pl.BlockSpec((n, k), lambda i: (0, 0))],
        out_specs=pl.BlockSpec((bo, n), lambda i: (i, 0)),
        out_shape=jax.ShapeDtypeStruct((m, n), jnp.float32),
    )(hs, hs, W)


# speedup vs baseline: 1.0159x; 1.0159x over previous
import jax
import jax.numpy as jnp
from jax.experimental import pallas as pl

_BM = 512


def _router_block(x_ref, w_ref, o_ref):
    o_ref[...] = jax.lax.dot_general(
        x_ref[...], w_ref[...],
        dimension_numbers=(((1,), (1,)), ((), ())),
        preferred_element_type=jnp.float32,
        precision=jax.lax.Precision.DEFAULT,
    )


def kernel(hidden_states, W):
    hs = hidden_states.reshape(-1, hidden_states.shape[-1])
    m, k = hs.shape
    n = W.shape[0]
    return pl.pallas_call(
        _router_block,
        grid=(m // _BM,),
        in_specs=[
            pl.BlockSpec((_BM, k), lambda i: (i, 0)),
            pl.BlockSpec((n, k), lambda i: (0, 0)),
        ],
        out_specs=pl.BlockSpec((_BM, n), lambda i: (i, 0)),
        out_shape=jax.ShapeDtypeStruct((m, n), jnp.float32),
    )(hs, W)
